# baseline (speedup 1.0000x reference)
import jax
import jax.numpy as jnp
from jax import lax
from jax.experimental import pallas as pl
from jax.experimental.pallas import tpu as pltpu

_TB = 256


def _acc_kernel(lens_ref, x_ref, tgt_ref, out_ref, c_ref, m_ref):
    b = pl.program_id(0)
    j = pl.program_id(1)
    nb = pl.num_programs(0)
    nj = pl.num_programs(1)
    x = x_ref[...]
    TB, V = x.shape
    rowmax = jnp.max(x, axis=-1, keepdims=True)
    idx = lax.broadcasted_iota(jnp.int32, x.shape, 1)
    pred = jnp.min(jnp.where(x == rowmax, idx, V), axis=-1)
    tgt = tgt_ref[0, 0]
    blen = lens_ref[b] + 1
    t_idx = lax.broadcasted_iota(jnp.int32, (TB,), 0) + j * TB
    mask = t_idx < blen
    c_part = jnp.sum(jnp.where(mask & (pred == tgt), 1.0, 0.0))
    m_part = jnp.sum(jnp.where(mask, 1.0, 0.0))

    @pl.when((b == 0) & (j == 0))
    def _init():
        c_ref[0] = 0.0
        m_ref[0] = 0.0

    @pl.when(j * TB < blen)
    def _acc():
        c_ref[0] += c_part
        m_ref[0] += m_part

    @pl.when((b == nb - 1) & (j == nj - 1))
    def _fin():
        out_ref[0] = c_ref[0] / m_ref[0]


def kernel(outputs, tokens, tokens_lens):
    T, B, V = outputs.shape
    nj = T // _TB
    x2d = outputs.reshape(T, B * V)
    tgt = jnp.concatenate([tokens[1:], tokens[-1:]], axis=0)
    tgt3 = tgt.T.reshape(B * nj, 1, _TB)

    def x_map(b, j, lens):
        nb_b = lax.div(lens[b] + _TB, _TB)
        return (jnp.minimum(j, nb_b - 1), b)

    def tgt_map(b, j, lens):
        nb_b = lax.div(lens[b] + _TB, _TB)
        return (b * nj + jnp.minimum(j, nb_b - 1), 0, 0)

    grid_spec = pltpu.PrefetchScalarGridSpec(
        num_scalar_prefetch=1,
        grid=(B, nj),
        in_specs=[
            pl.BlockSpec((_TB, V), x_map),
            pl.BlockSpec((1, 1, _TB), tgt_map),
        ],
        out_specs=pl.BlockSpec(memory_space=pltpu.SMEM),
        scratch_shapes=[
            pltpu.SMEM((1,), jnp.float32),
            pltpu.SMEM((1,), jnp.float32),
        ],
    )
    acc = pl.pallas_call(
        _acc_kernel,
        grid_spec=grid_spec,
        out_shape=jax.ShapeDtypeStruct((1,), jnp.float32),
        compiler_params=pltpu.CompilerParams(
            dimension_semantics=("arbitrary", "arbitrary"),
        ),
    )(tokens_lens, x2d, tgt3)
    return acc[0]
